# Initial kernel scaffold; baseline (speedup 1.0000x reference)
import jax
import jax.numpy as jnp
from jax.experimental import pallas as pl


def kernel(uids, iids, edge_index, edge_vals, uEmbed0, iEmbed0, uhyper, ihyper, WU, WI, WT):
    raise NotImplementedError("write your pallas kernel here")


# speedup vs baseline: 1.0020x; 1.0020x over previous
import functools

import jax
import jax.numpy as jnp
from jax.experimental import pallas as pl

N_USER = 10000
N_ITEM = 10000
LATDIM = 128
HYPERNUM = 128
GNN_LAYER = 2
N_EDGES = 320000
BATCH = 4096
LEAKY = 0.5
TEMP = 1.0


def _lr(x):
    return jnp.maximum(LEAKY * x, x)


def _normalize(x):
    n = jnp.linalg.norm(x, axis=1, keepdims=True)
    return x / jnp.maximum(n, 1e-12)


def _mm_kernel(x_ref, w_ref, o_ref):
    o_ref[...] = jnp.dot(x_ref[...], w_ref[...],
                         preferred_element_type=jnp.float32)


def _pallas_mm(x, w):
    m, k = x.shape
    k2, n = w.shape
    bm = 1000
    return pl.pallas_call(
        _mm_kernel,
        grid=(m // bm,),
        in_specs=[pl.BlockSpec((bm, k), lambda i: (i, 0)),
                  pl.BlockSpec((k, n), lambda i: (0, 0))],
        out_specs=pl.BlockSpec((bm, n), lambda i: (i, 0)),
        out_shape=jax.ShapeDtypeStruct((m, n), jnp.float32),
    )(x, w)


def _hyper_prop(lats, adj, W1, W2, W3):
    lat1 = _lr(adj.T @ lats)
    lat2 = _lr(lat1.T @ W1).T + lat1
    lat3 = _lr(lat2.T @ W2).T + lat2
    lat4 = _lr(lat3.T @ W3).T + lat3
    return _lr(adj @ lat4)


def _calc_ssl(h, g, m):
    pos = jnp.exp(jnp.sum(h * g, axis=1) / TEMP)
    neg = jnp.sum(jnp.exp(g @ h.T / TEMP) * m[None, :], axis=1)
    return jnp.sum(m * (-jnp.log(pos / (neg + 1e-08) + 1e-08)))


def kernel(uids, iids, edge_index, edge_vals, uEmbed0, iEmbed0, uhyper, ihyper, WU, WI, WT):
    row = edge_index[0]
    col = edge_index[1]
    uniq_u = jnp.unique(uids, size=BATCH, fill_value=0)
    uniq_i = jnp.unique(iids, size=BATCH, fill_value=0)
    present_u = jnp.zeros((N_USER,), dtype=jnp.bool_).at[uids].set(True)
    present_i = jnp.zeros((N_ITEM,), dtype=jnp.bool_).at[iids].set(True)
    num_u = jnp.sum(present_u)
    num_i = jnp.sum(present_i)
    mask_u = (jnp.arange(BATCH) < num_u).astype(jnp.float32)
    mask_i = (jnp.arange(BATCH) < num_i).astype(jnp.float32)

    uuHyper = _pallas_mm(uEmbed0, uhyper)
    iiHyper = _pallas_mm(iEmbed0, ihyper)
    ulats = [uEmbed0]
    ilats = [iEmbed0]
    gnnU, gnnI, hypU, hypI = [], [], [], []
    for i in range(GNN_LAYER):
        ulat = _lr(jax.ops.segment_sum(edge_vals[:, None] * ilats[-1][col], row, num_segments=N_USER))
        ilat = _lr(jax.ops.segment_sum(edge_vals[:, None] * ulats[-1][row], col, num_segments=N_ITEM))
        hU = _hyper_prop(ulats[-1], uuHyper, WU[i, 0], WU[i, 1], WU[i, 2])
        hI = _hyper_prop(ilats[-1], iiHyper, WI[i, 0], WI[i, 1], WI[i, 2])
        gnnU.append(ulat); gnnI.append(ilat); hypU.append(hU); hypI.append(hI)
        ulats.append(ulat + hU + ulats[-1])
        ilats.append(ilat + hI + ilats[-1])
    ulat = jnp.sum(jnp.stack(ulats), axis=0)
    ilat = jnp.sum(jnp.stack(ilats), axis=0)
    preds = jnp.sum(ulat[uids] * ilat[iids], axis=-1)
    ssl = 0.0
    for i in range(GNN_LAYER):
        pHU = _normalize(hypU[i][uniq_u]) @ WT[i]
        pGU = _normalize(gnnU[i][uniq_u])
        pHI = _normalize(hypI[i][uniq_i]) @ WT[i]
        pGI = _normalize(gnnI[i][uniq_i])
        ssl = ssl + _calc_ssl(pHU, pGU, mask_u) + _calc_ssl(pHI, pGI, mask_i)
    reg = (jnp.sum(jnp.square(uEmbed0)) + jnp.sum(jnp.square(iEmbed0))
           + jnp.sum(jnp.square(uhyper)) + jnp.sum(jnp.square(ihyper)))
    return (preds, ssl, reg)


# speedup vs baseline: 3.7932x; 3.7855x over previous
import functools

import jax
import jax.numpy as jnp
from jax import lax
from jax.experimental import pallas as pl
from jax.experimental.pallas import tpu as pltpu
from jax.experimental.pallas import tpu_sc as plsc

N_USER = 10000
N_ITEM = 10000
LATDIM = 128
HYPERNUM = 128
GNN_LAYER = 2
N_EDGES = 320000
BATCH = 4096
LEAKY = 0.5
TEMP = 1.0

NC = 2
NS = 16
L = 16

CHUNK = 128
NCHUNKS = N_EDGES // CHUNK
N_PAD = 10240
ROWS_PER_SUB = N_PAD // NS


def _lr(x):
    return jnp.maximum(LEAKY * x, x)


def _normalize(x):
    n = jnp.linalg.norm(x, axis=1, keepdims=True)
    return x / jnp.maximum(n, 1e-12)


def _spmm_body(tables, idx_src, idx_dst, vals, out,
               idx_g, idx_s, vals_v, rows_v, acc_sh, sem):
    c = lax.axis_index("c")
    s = lax.axis_index("s")

    def zero_rows(i, _):
        for k in range(8):
            rows_v[i, pl.ds(k * L, L)] = jnp.zeros((L,), jnp.float32)
        return 0
    lax.fori_loop(0, CHUNK, zero_rows, 0)
    base = s * ROWS_PER_SUB
    for r0 in range(0, ROWS_PER_SUB, CHUNK):
        pltpu.sync_copy(rows_v, acc_sh.at[pl.ds(base + r0, CHUNK)])
    plsc.subcore_barrier()

    nt = (NCHUNKS - s + NS - 1) // NS

    def chunk_body(t, _):
        chunk = s + t * NS
        pltpu.sync_copy(idx_src.at[c, chunk], idx_g)
        pltpu.sync_copy(idx_dst.at[c, chunk], idx_s)
        pltpu.sync_copy(vals.at[chunk], vals_v)
        pltpu.async_copy(tables.at[idx_g], rows_v, sem).wait()

        def scale_group(g, _):
            v16 = vals_v[pl.ds(g * L, L)]
            for j in range(L):
                vj = jnp.take(v16, jnp.full((L,), j, jnp.int32))
                e = g * L + j
                for k in range(8):
                    rows_v[e, pl.ds(k * L, L)] = rows_v[e, pl.ds(k * L, L)] * vj
            return 0
        lax.fori_loop(0, CHUNK // L, scale_group, 0)
        pltpu.sync_copy(rows_v, acc_sh.at[idx_s], add=True)
        return 0
    lax.fori_loop(0, nt, chunk_body, 0)
    plsc.subcore_barrier()

    pltpu.sync_copy(acc_sh.at[pl.ds(base, ROWS_PER_SUB)],
                    out.at[c, pl.ds(base, ROWS_PER_SUB)])


@jax.jit
def _sc_spmm(flat_tables, idx_src, idx_dst, vals):
    mesh = plsc.VectorSubcoreMesh(core_axis_name="c", subcore_axis_name="s")
    return pl.kernel(
        _spmm_body,
        out_type=jax.ShapeDtypeStruct((2, N_PAD, LATDIM), jnp.float32),
        mesh=mesh,
        scratch_types=[
            pltpu.VMEM((CHUNK,), jnp.int32),
            pltpu.VMEM((CHUNK,), jnp.int32),
            pltpu.VMEM((CHUNK,), jnp.float32),
            pltpu.VMEM((CHUNK, LATDIM), jnp.float32),
            pltpu.VMEM_SHARED((N_PAD, LATDIM), jnp.float32),
            pltpu.SemaphoreType.DMA,
        ],
    )(flat_tables, idx_src, idx_dst, vals)


def _mm_kernel(x_ref, w_ref, o_ref):
    o_ref[...] = jnp.dot(x_ref[...], w_ref[...],
                         preferred_element_type=jnp.float32)


def _pallas_mm(x, w):
    m, k = x.shape
    _, n = w.shape
    bm = 1000
    return pl.pallas_call(
        _mm_kernel,
        grid=(m // bm,),
        in_specs=[pl.BlockSpec((bm, k), lambda i: (i, 0)),
                  pl.BlockSpec((k, n), lambda i: (0, 0))],
        out_specs=pl.BlockSpec((bm, n), lambda i: (i, 0)),
        out_shape=jax.ShapeDtypeStruct((m, n), jnp.float32),
    )(x, w)


def _hyper_prop(lats, adj, W1, W2, W3):
    lat1 = _lr(adj.T @ lats)
    lat2 = _lr(lat1.T @ W1).T + lat1
    lat3 = _lr(lat2.T @ W2).T + lat2
    lat4 = _lr(lat3.T @ W3).T + lat3
    return _lr(adj @ lat4)


def _calc_ssl(h, g, m):
    pos = jnp.exp(jnp.sum(h * g, axis=1) / TEMP)
    neg = jnp.sum(jnp.exp(g @ h.T / TEMP) * m[None, :], axis=1)
    return jnp.sum(m * (-jnp.log(pos / (neg + 1e-08) + 1e-08)))


def kernel(uids, iids, edge_index, edge_vals, uEmbed0, iEmbed0, uhyper, ihyper, WU, WI, WT):
    row = edge_index[0]
    col = edge_index[1]
    uniq_u = jnp.unique(uids, size=BATCH, fill_value=0)
    uniq_i = jnp.unique(iids, size=BATCH, fill_value=0)
    present_u = jnp.zeros((N_USER,), dtype=jnp.bool_).at[uids].set(True)
    present_i = jnp.zeros((N_ITEM,), dtype=jnp.bool_).at[iids].set(True)
    num_u = jnp.sum(present_u)
    num_i = jnp.sum(present_i)
    mask_u = (jnp.arange(BATCH) < num_u).astype(jnp.float32)
    mask_i = (jnp.arange(BATCH) < num_i).astype(jnp.float32)

    idx_src = jnp.stack([col, row + N_USER]).reshape(2, NCHUNKS, CHUNK)
    idx_dst = jnp.stack([row, col]).reshape(2, NCHUNKS, CHUNK)
    vals_c = edge_vals.reshape(NCHUNKS, CHUNK)

    uuHyper = _pallas_mm(uEmbed0, uhyper)
    iiHyper = _pallas_mm(iEmbed0, ihyper)
    ulats = [uEmbed0]
    ilats = [iEmbed0]
    gnnU, gnnI, hypU, hypI = [], [], [], []
    for i in range(GNN_LAYER):
        flat_tables = jnp.concatenate([ilats[-1], ulats[-1]], axis=0)
        acc = _sc_spmm(flat_tables, idx_src, idx_dst, vals_c)
        ulat = _lr(acc[0, :N_USER])
        ilat = _lr(acc[1, :N_ITEM])
        hU = _hyper_prop(ulats[-1], uuHyper, WU[i, 0], WU[i, 1], WU[i, 2])
        hI = _hyper_prop(ilats[-1], iiHyper, WI[i, 0], WI[i, 1], WI[i, 2])
        gnnU.append(ulat); gnnI.append(ilat); hypU.append(hU); hypI.append(hI)
        ulats.append(ulat + hU + ulats[-1])
        ilats.append(ilat + hI + ilats[-1])
    ulat = jnp.sum(jnp.stack(ulats), axis=0)
    ilat = jnp.sum(jnp.stack(ilats), axis=0)
    preds = jnp.sum(ulat[uids] * ilat[iids], axis=-1)
    ssl = 0.0
    for i in range(GNN_LAYER):
        pHU = _normalize(hypU[i][uniq_u]) @ WT[i]
        pGU = _normalize(gnnU[i][uniq_u])
        pHI = _normalize(hypI[i][uniq_i]) @ WT[i]
        pGI = _normalize(gnnI[i][uniq_i])
        ssl = ssl + _calc_ssl(pHU, pGU, mask_u) + _calc_ssl(pHI, pGI, mask_i)
    reg = (jnp.sum(jnp.square(uEmbed0)) + jnp.sum(jnp.square(iEmbed0))
           + jnp.sum(jnp.square(uhyper)) + jnp.sum(jnp.square(ihyper)))
    return (preds, ssl, reg)


# speedup vs baseline: 5.0192x; 1.3232x over previous
import functools

import jax
import jax.numpy as jnp
from jax import lax
from jax.experimental import pallas as pl
from jax.experimental.pallas import tpu as pltpu
from jax.experimental.pallas import tpu_sc as plsc

N_USER = 10000
N_ITEM = 10000
LATDIM = 128
HYPERNUM = 128
GNN_LAYER = 2
N_EDGES = 320000
BATCH = 4096
LEAKY = 0.5
TEMP = 1.0

NC = 2
NS = 16
L = 16

CHUNK = 112
NT = 180
NCHUNKS = NS * NT
E_PAD = NCHUNKS * CHUNK
N_PAD = 10240
ROWS_PER_SUB = N_PAD // NS


def _lr(x):
    return jnp.maximum(LEAKY * x, x)


def _normalize(x):
    n = jnp.linalg.norm(x, axis=1, keepdims=True)
    return x / jnp.maximum(n, 1e-12)


def _spmm_body(tables, edp, vals, out,
               p0, p1, p2, v0, v1, v2, r0, r1, r2,
               g0, g1, g2, s0, s1, s2, acc_sh):
    c = lax.axis_index("c")
    s = lax.axis_index("s")
    packs = (p0, p1, p2)
    vbufs = (v0, v1, v2)
    rows = (r0, r1, r2)
    gsem = (g0, g1, g2)
    ssem = (s0, s1, s2)
    t0 = s * NT

    def zero_rows(i, _):
        for k in range(8):
            r0[i, pl.ds(k * L, L)] = jnp.zeros((L,), jnp.float32)
        return 0
    lax.fori_loop(0, CHUNK, zero_rows, 0)
    base = s * ROWS_PER_SUB
    for q in range(ROWS_PER_SUB // CHUNK):
        pltpu.sync_copy(r0, acc_sh.at[pl.ds(base + q * CHUNK, CHUNK)])
    pltpu.sync_copy(r0.at[pl.ds(0, ROWS_PER_SUB % CHUNK)],
                    acc_sh.at[pl.ds(base + (ROWS_PER_SUB // CHUNK) * CHUNK,
                                    ROWS_PER_SUB % CHUNK)])
    plsc.subcore_barrier()

    def i_copy(t, b):
        pltpu.sync_copy(edp.at[c, t0 + t], packs[b])
        pltpu.sync_copy(vals.at[t0 + t], vbufs[b])

    def g_start(t, b):
        pltpu.async_copy(tables.at[packs[b].at[0]], rows[b], gsem[b])

    def g_wait(t, b):
        pltpu.make_async_copy(tables.at[packs[b].at[0]], rows[b], gsem[b]).wait()

    def a_start(t, b):
        pltpu.async_copy(rows[b], acc_sh.at[packs[b].at[1]], ssem[b], add=True)

    def a_wait(t, b):
        pltpu.make_async_copy(rows[b], acc_sh.at[packs[b].at[1]], ssem[b]).wait()

    i_copy(0, 0)
    g_start(0, 0)

    def outer(q, _):
        for b in range(3):
            t = q * 3 + b
            bn = (b + 1) % 3
            @pl.when(t >= 2)
            def _():
                a_wait(t - 2, bn)

            @pl.when(t + 1 < NT)
            def _():
                i_copy(t + 1, bn)
                g_start(t + 1, bn)

            g_wait(t, b)

            def scale_group(g, _):
                v16 = vbufs[b][0, pl.ds(g * L, L)]
                rb = rows[b]
                for j in range(L):
                    vj = jnp.take(v16, jnp.full((L,), j, jnp.int32))
                    e = g * L + j
                    for k in range(8):
                        rb[e, pl.ds(k * L, L)] = rb[e, pl.ds(k * L, L)] * vj
                return 0
            lax.fori_loop(0, CHUNK // L, scale_group, 0)
            a_start(t, b)
        return 0
    lax.fori_loop(0, NT // 3, outer, 0)
    a_wait(NT - 2, (NT - 2) % 3)
    a_wait(NT - 1, (NT - 1) % 3)
    plsc.subcore_barrier()

    pltpu.sync_copy(acc_sh.at[pl.ds(base, ROWS_PER_SUB)],
                    out.at[c, pl.ds(base, ROWS_PER_SUB)])


@jax.jit
def _sc_spmm(flat_tables, edp, vals):
    mesh = plsc.VectorSubcoreMesh(core_axis_name="c", subcore_axis_name="s")
    return pl.kernel(
        _spmm_body,
        out_type=jax.ShapeDtypeStruct((2, N_PAD, LATDIM), jnp.float32),
        mesh=mesh,
        scratch_types=[
            pltpu.VMEM((2, CHUNK), jnp.int32),
            pltpu.VMEM((2, CHUNK), jnp.int32),
            pltpu.VMEM((2, CHUNK), jnp.int32),
            pltpu.VMEM((1, CHUNK), jnp.float32),
            pltpu.VMEM((1, CHUNK), jnp.float32),
            pltpu.VMEM((1, CHUNK), jnp.float32),
            pltpu.VMEM((CHUNK, LATDIM), jnp.float32),
            pltpu.VMEM((CHUNK, LATDIM), jnp.float32),
            pltpu.VMEM((CHUNK, LATDIM), jnp.float32),
            pltpu.SemaphoreType.DMA,
            pltpu.SemaphoreType.DMA,
            pltpu.SemaphoreType.DMA,
            pltpu.SemaphoreType.DMA,
            pltpu.SemaphoreType.DMA,
            pltpu.SemaphoreType.DMA,
            pltpu.VMEM_SHARED((N_PAD, LATDIM), jnp.float32),
        ],
    )(flat_tables, edp, vals)


def _mm_kernel(x_ref, w_ref, o_ref):
    o_ref[...] = jnp.dot(x_ref[...], w_ref[...],
                         preferred_element_type=jnp.float32)


def _pallas_mm(x, w):
    m, k = x.shape
    _, n = w.shape
    bm = 1000
    return pl.pallas_call(
        _mm_kernel,
        grid=(m // bm,),
        in_specs=[pl.BlockSpec((bm, k), lambda i: (i, 0)),
                  pl.BlockSpec((k, n), lambda i: (0, 0))],
        out_specs=pl.BlockSpec((bm, n), lambda i: (i, 0)),
        out_shape=jax.ShapeDtypeStruct((m, n), jnp.float32),
    )(x, w)


def _hyper_prop(lats, adj, W1, W2, W3):
    lat1 = _lr(adj.T @ lats)
    lat2 = _lr(lat1.T @ W1).T + lat1
    lat3 = _lr(lat2.T @ W2).T + lat2
    lat4 = _lr(lat3.T @ W3).T + lat3
    return _lr(adj @ lat4)


def _calc_ssl(h, g, m):
    pos = jnp.exp(jnp.sum(h * g, axis=1) / TEMP)
    neg = jnp.sum(jnp.exp(g @ h.T / TEMP) * m[None, :], axis=1)
    return jnp.sum(m * (-jnp.log(pos / (neg + 1e-08) + 1e-08)))


def kernel(uids, iids, edge_index, edge_vals, uEmbed0, iEmbed0, uhyper, ihyper, WU, WI, WT):
    row = edge_index[0]
    col = edge_index[1]
    uniq_u = jnp.unique(uids, size=BATCH, fill_value=0)
    uniq_i = jnp.unique(iids, size=BATCH, fill_value=0)
    present_u = jnp.zeros((N_USER,), dtype=jnp.bool_).at[uids].set(True)
    present_i = jnp.zeros((N_ITEM,), dtype=jnp.bool_).at[iids].set(True)
    num_u = jnp.sum(present_u)
    num_i = jnp.sum(present_i)
    mask_u = (jnp.arange(BATCH) < num_u).astype(jnp.float32)
    mask_i = (jnp.arange(BATCH) < num_i).astype(jnp.float32)

    padz = jnp.zeros((E_PAD - N_EDGES,), dtype=jnp.int32)
    colp = jnp.concatenate([col, padz]).reshape(NCHUNKS, CHUNK)
    rowp = jnp.concatenate([row, padz]).reshape(NCHUNKS, CHUNK)
    valp = jnp.concatenate([edge_vals, padz.astype(jnp.float32)]
                           ).reshape(NCHUNKS, 1, CHUNK)
    edp = jnp.stack([
        jnp.stack([colp, rowp], axis=1),
        jnp.stack([rowp + N_USER, colp], axis=1),
    ])

    uuHyper = _pallas_mm(uEmbed0, uhyper)
    iiHyper = _pallas_mm(iEmbed0, ihyper)
    ulats = [uEmbed0]
    ilats = [iEmbed0]
    gnnU, gnnI, hypU, hypI = [], [], [], []
    for i in range(GNN_LAYER):
        flat_tables = jnp.concatenate([ilats[-1], ulats[-1]], axis=0)
        acc = _sc_spmm(flat_tables, edp, valp)
        ulat = _lr(acc[0, :N_USER])
        ilat = _lr(acc[1, :N_ITEM])
        hU = _hyper_prop(ulats[-1], uuHyper, WU[i, 0], WU[i, 1], WU[i, 2])
        hI = _hyper_prop(ilats[-1], iiHyper, WI[i, 0], WI[i, 1], WI[i, 2])
        gnnU.append(ulat); gnnI.append(ilat); hypU.append(hU); hypI.append(hI)
        ulats.append(ulat + hU + ulats[-1])
        ilats.append(ilat + hI + ilats[-1])
    ulat = jnp.sum(jnp.stack(ulats), axis=0)
    ilat = jnp.sum(jnp.stack(ilats), axis=0)
    preds = jnp.sum(ulat[uids] * ilat[iids], axis=-1)
    ssl = 0.0
    for i in range(GNN_LAYER):
        pHU = _normalize(hypU[i][uniq_u]) @ WT[i]
        pGU = _normalize(gnnU[i][uniq_u])
        pHI = _normalize(hypI[i][uniq_i]) @ WT[i]
        pGI = _normalize(gnnI[i][uniq_i])
        ssl = ssl + _calc_ssl(pHU, pGU, mask_u) + _calc_ssl(pHI, pGI, mask_i)
    reg = (jnp.sum(jnp.square(uEmbed0)) + jnp.sum(jnp.square(iEmbed0))
           + jnp.sum(jnp.square(uhyper)) + jnp.sum(jnp.square(ihyper)))
    return (preds, ssl, reg)
